# baseline (speedup 1.0000x reference)
import functools

import jax
import jax.numpy as jnp
from jax import lax
from jax.experimental import pallas as pl
from jax.experimental.pallas import tpu as pltpu
from jax.experimental.pallas import tpu_sc as plsc

_F32 = jnp.float32
_I32 = jnp.int32


def _prep1_body(rot_ref, t_ref, a12_ref):
    q = rot_ref[...]
    norm = jnp.sqrt(jnp.sum(q * q, axis=1, keepdims=True))
    qn = q / jnp.maximum(norm, 1e-8)
    qw = qn[:, 0:1]
    qx = qn[:, 1:2]
    qy = qn[:, 2:3]
    qz = qn[:, 3:4]
    xx = qx * qx
    yy = qy * qy
    zz = qz * qz
    xy = qx * qy
    xz = qx * qz
    yz = qy * qz
    wx = qw * qx
    wy = qw * qy
    wz = qw * qz
    one = jnp.ones_like(qw)
    r00 = one - 2.0 * (yy + zz)
    r01 = 2.0 * (xy + wz)
    r02 = 2.0 * (xz - wy)
    r10 = 2.0 * (xy - wz)
    r11 = one - 2.0 * (xx + zz)
    r12 = 2.0 * (yz + wx)
    r20 = 2.0 * (xz + wy)
    r21 = 2.0 * (yz - wx)
    r22 = one - 2.0 * (xx + yy)
    t = t_ref[...]
    tx = t[:, 0:1]
    ty = t[:, 1:2]
    tz = t[:, 2:3]
    c0 = -(r00 * tx + r01 * ty + r02 * tz)
    c1 = -(r10 * tx + r11 * ty + r12 * tz)
    c2 = -(r20 * tx + r21 * ty + r22 * tz)
    a12_ref[...] = jnp.concatenate(
        [r00, r10, r20, r01, r11, r21, r02, r12, r22, c0, c1, c2], axis=1)


def _prep2_body(abig_ref, wp_ref, feat_ref, wc_ref, bc_ref, bp_ref,
                g_ref, bias_ref):
    g_ref[...] = jnp.dot(abig_ref[...], wp_ref[...],
                         preferred_element_type=_F32)
    bias_ref[...] = (
        jnp.dot(feat_ref[...], wc_ref[...], preferred_element_type=_F32)
        + bc_ref[...] + bp_ref[...])


def _ptm_body(paug_ref, a_ref, ptm_ref):
    ptm_ref[0] = jnp.dot(paug_ref[0], a_ref[0], preferred_element_type=_F32)


def _sc_compact_body(mask_hbm, paug_hbm, idx_hbm, cp_hbm, cnt_hbm,
                     mask_v, paug_v, idx_v, cp_v, cnt_v, *, Q, M):
    c = lax.axis_index("c")
    s = lax.axis_index("s")
    seg = c * 8 + s

    @pl.when(s < 8)
    def _():
        b = seg // M
        pltpu.sync_copy(mask_hbm.at[seg], mask_v)
        pltpu.sync_copy(paug_hbm.at[b], paug_v)
        sentinel = jnp.full((16,), Q, _I32)

        def pre(j, carry):
            idx_v[pl.ds(j * 16, 16)] = sentinel
            return carry

        lax.fori_loop(0, (Q + 16) // 16, pre, 0)

        def cmp(j, off):
            mv = mask_v[pl.ds(j * 16, 16)]
            qi = lax.iota(_I32, 16) + j * 16
            pos = off + plsc.cumsum(mv) - mv
            plsc.store_scatter(idx_v, [pos], qi, mask=mv > 0)
            return off + jnp.sum(mv)

        k = lax.fori_loop(0, Q // 16, cmp, jnp.int32(0))
        cnt_v[...] = jnp.zeros((16,), _I32) + k
        pltpu.sync_copy(cnt_v, cnt_hbm.at[seg])

        ones = jnp.ones((16,), _F32)

        def gat(j, carry):
            iv = idx_v[pl.ds(j * 16, 16)]
            base = iv * 4
            rowi = (lax.iota(_I32, 16) + j * 16) * 4
            for ci in range(3):
                comp = plsc.load_gather(paug_v, [base + ci])
                plsc.store_scatter(cp_v, [rowi + ci], comp)
            plsc.store_scatter(cp_v, [rowi + 3], ones)
            return carry

        lax.fori_loop(0, Q // 16, gat, 0)
        pltpu.sync_copy(idx_v.at[pl.ds(0, Q)], idx_hbm.at[seg])
        pltpu.sync_copy(cp_v, cp_hbm.at[seg])


def _mlp_body(cnt_ref, cp_ref, g_ref, bias_ref, w1_ref, b1_ref, w2_ref,
              b2_ref, wout_ref, bout_ref, out_ref, *, T):
    s = pl.program_id(0)
    i = pl.program_id(1)

    @pl.when(i * T < cnt_ref[s])
    def _():
        x = cp_ref[0]
        net = jnp.dot(x, g_ref[0], preferred_element_type=_F32) + bias_ref[0]
        h = jnp.dot(jnp.maximum(net, 0.0), w1_ref[...],
                    preferred_element_type=_F32) + b1_ref[...]
        h = jnp.dot(jnp.maximum(h, 0.0), w2_ref[...],
                    preferred_element_type=_F32) + b2_ref[...]
        net = net + h
        occ = (jnp.sum(jnp.maximum(net, 0.0) * wout_ref[...],
                       axis=1, keepdims=True) + bout_ref[...])
        out_ref[0, 0] = jax.nn.sigmoid(10.0 * occ).T


def _sc_scatter_body(sig_hbm, idx_hbm, imp_hbm, sig_v, idx_v, out_v, *, Q):
    c = lax.axis_index("c")
    s = lax.axis_index("s")
    seg = c * 8 + s

    @pl.when(s < 8)
    def _():
        pltpu.sync_copy(sig_hbm.at[seg], sig_v)
        pltpu.sync_copy(idx_hbm.at[seg], idx_v)
        zero = jnp.zeros((16,), _F32)

        def zer(j, carry):
            out_v[pl.ds(j * 16, 16)] = zero
            return carry

        lax.fori_loop(0, (Q + 16) // 16, zer, 0)

        def sca(j, carry):
            iv = idx_v[pl.ds(j * 16, 16)]
            sv = sig_v[pl.ds(j * 16, 16)]
            plsc.store_scatter(out_v, [iv], sv)
            return carry

        lax.fori_loop(0, Q // 16, sca, 0)
        pltpu.sync_copy(out_v.at[pl.ds(0, Q)], imp_hbm.at[seg])


def kernel(ray_points, translations, rotations, part_shape_features,
           points_mask, W_p, b_p, W_c, b_c, W1, b1, W2, b2, W_out, b_out):
    B, N, P, _ = ray_points.shape
    M = translations.shape[1]
    C = part_shape_features.shape[-1]
    H = W_p.shape[1]
    Q = N * P
    S = B * M

    a12 = pl.pallas_call(
        _prep1_body,
        out_shape=jax.ShapeDtypeStruct((S, 12), _F32),
    )(rotations.reshape(S, 4), translations.reshape(S, 3))

    abig = a12.reshape(S * 4, 3)
    a = a12.reshape(B, M, 4, 3).transpose(0, 2, 1, 3).reshape(B, 4, M * 3)

    g, bias = pl.pallas_call(
        _prep2_body,
        out_shape=(
            jax.ShapeDtypeStruct((S * 4, H), _F32),
            jax.ShapeDtypeStruct((S, H), _F32),
        ),
    )(abig, W_p, part_shape_features.reshape(S, C), W_c,
      b_c.reshape(1, H), b_p.reshape(1, H))

    pts = ray_points.reshape(B, Q, 3)
    paug = jnp.concatenate([pts, jnp.ones((B, Q, 1), _F32)], axis=-1)

    TP = 2048
    ptm = pl.pallas_call(
        _ptm_body,
        grid=(B, Q // TP),
        in_specs=[
            pl.BlockSpec((1, TP, 4), lambda b, i: (b, i, 0)),
            pl.BlockSpec((1, 4, M * 3), lambda b, i: (b, 0, 0)),
        ],
        out_specs=pl.BlockSpec((1, TP, M * 3), lambda b, i: (b, i, 0)),
        out_shape=jax.ShapeDtypeStruct((B, Q, M * 3), _F32),
        compiler_params=pltpu.CompilerParams(
            dimension_semantics=("parallel", "parallel")),
    )(paug, a)

    mask_t = (points_mask.reshape(B, Q, M).transpose(0, 2, 1)
              .reshape(S, Q).astype(_I32))
    paug_pad = jnp.concatenate(
        [paug, jnp.zeros((B, 16, 4), _F32)], axis=1).reshape(B, (Q + 16) * 4)

    mesh = plsc.VectorSubcoreMesh(core_axis_name="c", subcore_axis_name="s")
    idx, cp, cnt = pl.kernel(
        functools.partial(_sc_compact_body, Q=Q, M=M),
        out_type=(
            jax.ShapeDtypeStruct((S, Q), _I32),
            jax.ShapeDtypeStruct((S, Q * 4), _F32),
            jax.ShapeDtypeStruct((S, 16), _I32),
        ),
        mesh=mesh,
        scratch_types=[
            pltpu.VMEM((Q,), _I32),
            pltpu.VMEM(((Q + 16) * 4,), _F32),
            pltpu.VMEM((Q + 16,), _I32),
            pltpu.VMEM((Q * 4,), _F32),
            pltpu.VMEM((16,), _I32),
        ],
        compiler_params=pltpu.CompilerParams(needs_layout_passes=False),
    )(mask_t, paug_pad)

    counts = cnt[:, 0]

    T = 512
    nblk = Q // T
    sig = pl.pallas_call(
        functools.partial(_mlp_body, T=T),
        grid_spec=pltpu.PrefetchScalarGridSpec(
            num_scalar_prefetch=1,
            grid=(S, nblk),
            in_specs=[
                pl.BlockSpec((1, T, 4), lambda s, i, cnt: (s, i, 0)),
                pl.BlockSpec((1, 4, H), lambda s, i, cnt: (s, 0, 0)),
                pl.BlockSpec((1, 1, H), lambda s, i, cnt: (s, 0, 0)),
                pl.BlockSpec((H, H), lambda s, i, cnt: (0, 0)),
                pl.BlockSpec((1, H), lambda s, i, cnt: (0, 0)),
                pl.BlockSpec((H, H), lambda s, i, cnt: (0, 0)),
                pl.BlockSpec((1, H), lambda s, i, cnt: (0, 0)),
                pl.BlockSpec((1, H), lambda s, i, cnt: (0, 0)),
                pl.BlockSpec((1, 1), lambda s, i, cnt: (0, 0)),
            ],
            out_specs=pl.BlockSpec((1, 1, 1, T),
                                   lambda s, i, cnt: (s, i, 0, 0)),
        ),
        out_shape=jax.ShapeDtypeStruct((S, nblk, 1, T), _F32),
        compiler_params=pltpu.CompilerParams(
            dimension_semantics=("arbitrary", "arbitrary")),
    )(counts, cp.reshape(S, Q, 4), g.reshape(S, 4, H), bias.reshape(S, 1, H),
      W1, b1.reshape(1, H), W2, b2.reshape(1, H), W_out.reshape(1, H),
      b_out.reshape(1, 1))

    imp_t = pl.kernel(
        functools.partial(_sc_scatter_body, Q=Q),
        out_type=jax.ShapeDtypeStruct((S, Q), _F32),
        mesh=plsc.VectorSubcoreMesh(core_axis_name="c",
                                    subcore_axis_name="s"),
        scratch_types=[
            pltpu.VMEM((Q,), _F32),
            pltpu.VMEM((Q,), _I32),
            pltpu.VMEM((Q + 16,), _F32),
        ],
        compiler_params=pltpu.CompilerParams(needs_layout_passes=False),
    )(sig.reshape(S, Q), idx)

    implicit_field = (imp_t.reshape(B, M, Q).transpose(0, 2, 1)
                      .reshape(B, N, P, M))
    points_transformed = ptm.reshape(B, N, P, M * 3)
    return implicit_field, points_transformed


# speedup vs baseline: 1.2539x; 1.2539x over previous
import functools

import jax
import jax.numpy as jnp
from jax import lax
from jax.experimental import pallas as pl
from jax.experimental.pallas import tpu as pltpu
from jax.experimental.pallas import tpu_sc as plsc

_F32 = jnp.float32
_I32 = jnp.int32


def _prep1_body(rot_ref, t_ref, a12_ref):
    q = rot_ref[...]
    norm = jnp.sqrt(jnp.sum(q * q, axis=1, keepdims=True))
    qn = q / jnp.maximum(norm, 1e-8)
    qw = qn[:, 0:1]
    qx = qn[:, 1:2]
    qy = qn[:, 2:3]
    qz = qn[:, 3:4]
    xx = qx * qx
    yy = qy * qy
    zz = qz * qz
    xy = qx * qy
    xz = qx * qz
    yz = qy * qz
    wx = qw * qx
    wy = qw * qy
    wz = qw * qz
    one = jnp.ones_like(qw)
    r00 = one - 2.0 * (yy + zz)
    r01 = 2.0 * (xy + wz)
    r02 = 2.0 * (xz - wy)
    r10 = 2.0 * (xy - wz)
    r11 = one - 2.0 * (xx + zz)
    r12 = 2.0 * (yz + wx)
    r20 = 2.0 * (xz + wy)
    r21 = 2.0 * (yz - wx)
    r22 = one - 2.0 * (xx + yy)
    t = t_ref[...]
    tx = t[:, 0:1]
    ty = t[:, 1:2]
    tz = t[:, 2:3]
    c0 = -(r00 * tx + r01 * ty + r02 * tz)
    c1 = -(r10 * tx + r11 * ty + r12 * tz)
    c2 = -(r20 * tx + r21 * ty + r22 * tz)
    a12_ref[...] = jnp.concatenate(
        [r00, r10, r20, r01, r11, r21, r02, r12, r22, c0, c1, c2], axis=1)


def _prep2_body(abig_ref, wp_ref, feat_ref, wc_ref, bc_ref, bp_ref,
                g_ref, bias_ref):
    g_ref[...] = jnp.dot(abig_ref[...], wp_ref[...],
                         preferred_element_type=_F32)
    bias_ref[...] = (
        jnp.dot(feat_ref[...], wc_ref[...], preferred_element_type=_F32)
        + bc_ref[...] + bp_ref[...])


def _ptm_body(paug_ref, a_ref, ptm_ref):
    ptm_ref[0] = jnp.dot(paug_ref[0], a_ref[0], preferred_element_type=_F32)


def _sc_compact_body(mask_hbm, paug_hbm, idx_hbm, cp_hbm, cnt_hbm,
                     mask_v, paug_v, idx_v, cp_v, cnt_v, *, Q, M):
    c = lax.axis_index("c")
    s = lax.axis_index("s")
    seg = c * 8 + s

    @pl.when(s < 8)
    def _():
        b = seg // M
        pltpu.sync_copy(mask_hbm.at[seg], mask_v)
        pltpu.sync_copy(paug_hbm.at[b], paug_v)
        sentinel = jnp.full((16,), Q, _I32)

        def pre(j, carry):
            idx_v[pl.ds(j * 16, 16)] = sentinel
            return carry

        lax.fori_loop(0, (Q + 16) // 16, pre, 0)

        def cmp(j, off):
            mv = mask_v[pl.ds(j * 16, 16)]
            qi = lax.iota(_I32, 16) + j * 16
            pos = off + plsc.cumsum(mv) - mv
            plsc.store_scatter(idx_v, [pos], qi, mask=mv > 0)
            return off + jnp.sum(mv)

        k = lax.fori_loop(0, Q // 16, cmp, jnp.int32(0))
        cnt_v[...] = jnp.zeros((16,), _I32) + k
        pltpu.sync_copy(cnt_v, cnt_hbm.at[seg])

        ones = jnp.ones((16,), _F32)

        def gat(j, carry):
            iv = idx_v[pl.ds(j * 16, 16)]
            base = iv * 4
            rowi = (lax.iota(_I32, 16) + j * 16) * 4
            for ci in range(3):
                comp = plsc.load_gather(paug_v, [base + ci])
                plsc.store_scatter(cp_v, [rowi + ci], comp)
            plsc.store_scatter(cp_v, [rowi + 3], ones)
            return carry

        lax.fori_loop(0, Q // 16, gat, 0)
        pltpu.sync_copy(idx_v.at[pl.ds(0, Q)], idx_hbm.at[seg])
        pltpu.sync_copy(cp_v, cp_hbm.at[seg])


def _mlp_body(cnt_ref, cp_ref, g_ref, bias_ref, w1_ref, b1_ref, w2_ref,
              b2_ref, wout_ref, bout_ref, out_ref, *, T):
    s = pl.program_id(0)
    cnt = cnt_ref[s]
    nblk = (cnt + T - 1) // T
    g = g_ref[0]
    bias = bias_ref[0]
    w1 = w1_ref[...]
    b1 = b1_ref[...]
    w2 = w2_ref[...]
    b2 = b2_ref[...]
    wout = wout_ref[...]
    bout = bout_ref[...]

    def step(i, carry):
        x = cp_ref[0, pl.ds(i * T, T), :]
        net = jnp.dot(x, g, preferred_element_type=_F32) + bias
        h = jnp.dot(jnp.maximum(net, 0.0), w1,
                    preferred_element_type=_F32) + b1
        h = jnp.dot(jnp.maximum(h, 0.0), w2,
                    preferred_element_type=_F32) + b2
        net = net + h
        occ = (jnp.sum(jnp.maximum(net, 0.0) * wout,
                       axis=1, keepdims=True) + bout)
        out_ref[0, :, pl.ds(i * T, T)] = jax.nn.sigmoid(10.0 * occ).T
        return carry

    lax.fori_loop(0, nblk, step, 0)


def _sc_scatter_body(sig_hbm, idx_hbm, imp_hbm, sig_v, idx_v, out_v, *, Q):
    c = lax.axis_index("c")
    s = lax.axis_index("s")
    seg = c * 8 + s

    @pl.when(s < 8)
    def _():
        pltpu.sync_copy(sig_hbm.at[seg], sig_v)
        pltpu.sync_copy(idx_hbm.at[seg], idx_v)
        zero = jnp.zeros((16,), _F32)

        def zer(j, carry):
            out_v[pl.ds(j * 16, 16)] = zero
            return carry

        lax.fori_loop(0, (Q + 16) // 16, zer, 0)

        def sca(j, carry):
            iv = idx_v[pl.ds(j * 16, 16)]
            sv = sig_v[pl.ds(j * 16, 16)]
            plsc.store_scatter(out_v, [iv], sv)
            return carry

        lax.fori_loop(0, Q // 16, sca, 0)
        pltpu.sync_copy(out_v.at[pl.ds(0, Q)], imp_hbm.at[seg])


def kernel(ray_points, translations, rotations, part_shape_features,
           points_mask, W_p, b_p, W_c, b_c, W1, b1, W2, b2, W_out, b_out):
    B, N, P, _ = ray_points.shape
    M = translations.shape[1]
    C = part_shape_features.shape[-1]
    H = W_p.shape[1]
    Q = N * P
    S = B * M

    a12 = pl.pallas_call(
        _prep1_body,
        out_shape=jax.ShapeDtypeStruct((S, 12), _F32),
    )(rotations.reshape(S, 4), translations.reshape(S, 3))

    abig = a12.reshape(S * 4, 3)
    a = a12.reshape(B, M, 4, 3).transpose(0, 2, 1, 3).reshape(B, 4, M * 3)

    g, bias = pl.pallas_call(
        _prep2_body,
        out_shape=(
            jax.ShapeDtypeStruct((S * 4, H), _F32),
            jax.ShapeDtypeStruct((S, H), _F32),
        ),
    )(abig, W_p, part_shape_features.reshape(S, C), W_c,
      b_c.reshape(1, H), b_p.reshape(1, H))

    pts = ray_points.reshape(B, Q, 3)
    paug = jnp.concatenate([pts, jnp.ones((B, Q, 1), _F32)], axis=-1)

    TP = 2048
    ptm = pl.pallas_call(
        _ptm_body,
        grid=(B, Q // TP),
        in_specs=[
            pl.BlockSpec((1, TP, 4), lambda b, i: (b, i, 0)),
            pl.BlockSpec((1, 4, M * 3), lambda b, i: (b, 0, 0)),
        ],
        out_specs=pl.BlockSpec((1, TP, M * 3), lambda b, i: (b, i, 0)),
        out_shape=jax.ShapeDtypeStruct((B, Q, M * 3), _F32),
        compiler_params=pltpu.CompilerParams(
            dimension_semantics=("parallel", "parallel")),
    )(paug, a)

    mask_t = (points_mask.reshape(B, Q, M).transpose(0, 2, 1)
              .reshape(S, Q).astype(_I32))
    paug_pad = jnp.concatenate(
        [paug, jnp.zeros((B, 16, 4), _F32)], axis=1).reshape(B, (Q + 16) * 4)

    mesh = plsc.VectorSubcoreMesh(core_axis_name="c", subcore_axis_name="s")
    idx, cp, cnt = pl.kernel(
        functools.partial(_sc_compact_body, Q=Q, M=M),
        out_type=(
            jax.ShapeDtypeStruct((S, Q), _I32),
            jax.ShapeDtypeStruct((S, Q * 4), _F32),
            jax.ShapeDtypeStruct((S, 16), _I32),
        ),
        mesh=mesh,
        scratch_types=[
            pltpu.VMEM((Q,), _I32),
            pltpu.VMEM(((Q + 16) * 4,), _F32),
            pltpu.VMEM((Q + 16,), _I32),
            pltpu.VMEM((Q * 4,), _F32),
            pltpu.VMEM((16,), _I32),
        ],
        compiler_params=pltpu.CompilerParams(needs_layout_passes=False),
    )(mask_t, paug_pad)

    counts = cnt[:, 0]

    T = 512
    sig = pl.pallas_call(
        functools.partial(_mlp_body, T=T),
        grid_spec=pltpu.PrefetchScalarGridSpec(
            num_scalar_prefetch=1,
            grid=(S,),
            in_specs=[
                pl.BlockSpec((1, Q, 4), lambda s, cnt: (s, 0, 0)),
                pl.BlockSpec((1, 4, H), lambda s, cnt: (s, 0, 0)),
                pl.BlockSpec((1, 1, H), lambda s, cnt: (s, 0, 0)),
                pl.BlockSpec((H, H), lambda s, cnt: (0, 0)),
                pl.BlockSpec((1, H), lambda s, cnt: (0, 0)),
                pl.BlockSpec((H, H), lambda s, cnt: (0, 0)),
                pl.BlockSpec((1, H), lambda s, cnt: (0, 0)),
                pl.BlockSpec((1, H), lambda s, cnt: (0, 0)),
                pl.BlockSpec((1, 1), lambda s, cnt: (0, 0)),
            ],
            out_specs=pl.BlockSpec((1, 1, Q), lambda s, cnt: (s, 0, 0)),
        ),
        out_shape=jax.ShapeDtypeStruct((S, 1, Q), _F32),
        compiler_params=pltpu.CompilerParams(
            dimension_semantics=("arbitrary",)),
    )(counts, cp.reshape(S, Q, 4), g.reshape(S, 4, H), bias.reshape(S, 1, H),
      W1, b1.reshape(1, H), W2, b2.reshape(1, H), W_out.reshape(1, H),
      b_out.reshape(1, 1))

    imp_t = pl.kernel(
        functools.partial(_sc_scatter_body, Q=Q),
        out_type=jax.ShapeDtypeStruct((S, Q), _F32),
        mesh=plsc.VectorSubcoreMesh(core_axis_name="c",
                                    subcore_axis_name="s"),
        scratch_types=[
            pltpu.VMEM((Q,), _F32),
            pltpu.VMEM((Q,), _I32),
            pltpu.VMEM((Q + 16,), _F32),
        ],
        compiler_params=pltpu.CompilerParams(needs_layout_passes=False),
    )(sig.reshape(S, Q), idx)

    implicit_field = (imp_t.reshape(B, M, Q).transpose(0, 2, 1)
                      .reshape(B, N, P, M))
    points_transformed = ptm.reshape(B, N, P, M * 3)
    return implicit_field, points_transformed


# speedup vs baseline: 1.3725x; 1.0946x over previous
import functools

import jax
import jax.numpy as jnp
from jax import lax
from jax.experimental import pallas as pl
from jax.experimental.pallas import tpu as pltpu
from jax.experimental.pallas import tpu_sc as plsc

_F32 = jnp.float32
_I32 = jnp.int32


def _prep1_body(rot_ref, t_ref, a12_ref):
    q = rot_ref[...]
    norm = jnp.sqrt(jnp.sum(q * q, axis=1, keepdims=True))
    qn = q / jnp.maximum(norm, 1e-8)
    qw = qn[:, 0:1]
    qx = qn[:, 1:2]
    qy = qn[:, 2:3]
    qz = qn[:, 3:4]
    xx = qx * qx
    yy = qy * qy
    zz = qz * qz
    xy = qx * qy
    xz = qx * qz
    yz = qy * qz
    wx = qw * qx
    wy = qw * qy
    wz = qw * qz
    one = jnp.ones_like(qw)
    r00 = one - 2.0 * (yy + zz)
    r01 = 2.0 * (xy + wz)
    r02 = 2.0 * (xz - wy)
    r10 = 2.0 * (xy - wz)
    r11 = one - 2.0 * (xx + zz)
    r12 = 2.0 * (yz + wx)
    r20 = 2.0 * (xz + wy)
    r21 = 2.0 * (yz - wx)
    r22 = one - 2.0 * (xx + yy)
    t = t_ref[...]
    tx = t[:, 0:1]
    ty = t[:, 1:2]
    tz = t[:, 2:3]
    c0 = -(r00 * tx + r01 * ty + r02 * tz)
    c1 = -(r10 * tx + r11 * ty + r12 * tz)
    c2 = -(r20 * tx + r21 * ty + r22 * tz)
    a12_ref[...] = jnp.concatenate(
        [r00, r10, r20, r01, r11, r21, r02, r12, r22, c0, c1, c2], axis=1)


def _prep2_body(abig_ref, wp_ref, feat_ref, wc_ref, bc_ref, bp_ref,
                g_ref, bias_ref):
    g_ref[...] = jnp.dot(abig_ref[...], wp_ref[...],
                         preferred_element_type=_F32)
    bias_ref[...] = (
        jnp.dot(feat_ref[...], wc_ref[...], preferred_element_type=_F32)
        + bc_ref[...] + bp_ref[...])


def _ptm_body(paug_ref, a_ref, ptm_ref):
    ptm_ref[0] = jnp.dot(paug_ref[0], a_ref[0], preferred_element_type=_F32)


def _sc_compact_body(mask_hbm, paug_hbm, idx_hbm, cp_hbm, cnt_hbm,
                     mask_v, paug_v, idx_v, cp_v, cnt_v, *, Q, M):
    c = lax.axis_index("c")
    s = lax.axis_index("s")
    seg = c * 8 + s

    @pl.when(s < 8)
    def _():
        b = seg // M
        pltpu.sync_copy(mask_hbm.at[seg], mask_v)
        pltpu.sync_copy(paug_hbm.at[b], paug_v)
        sentinel = jnp.full((16,), Q, _I32)

        def pre(j, carry):
            idx_v[pl.ds(j * 16, 16)] = sentinel
            return carry

        lax.fori_loop(0, (Q + 16) // 16, pre, 0)

        def cmp(j, off):
            mv = mask_v[pl.ds(j * 16, 16)]
            qi = lax.iota(_I32, 16) + j * 16
            pos = off + plsc.cumsum(mv) - mv
            plsc.store_scatter(idx_v, [pos], qi, mask=mv > 0)
            return off + jnp.sum(mv)

        k = lax.fori_loop(0, Q // 16, cmp, jnp.int32(0))
        cnt_v[...] = jnp.zeros((16,), _I32) + k
        pltpu.sync_copy(cnt_v, cnt_hbm.at[seg])

        ones = jnp.ones((16,), _F32)

        def gat(j, carry):
            iv = idx_v[pl.ds(j * 16, 16)]
            base = iv * 4
            rowi = (lax.iota(_I32, 16) + j * 16) * 4
            for ci in range(3):
                comp = plsc.load_gather(paug_v, [base + ci])
                plsc.store_scatter(cp_v, [rowi + ci], comp)
            plsc.store_scatter(cp_v, [rowi + 3], ones)
            return carry

        lax.fori_loop(0, Q // 16, gat, 0)
        pltpu.sync_copy(idx_v.at[pl.ds(0, Q)], idx_hbm.at[seg])
        pltpu.sync_copy(cp_v, cp_hbm.at[seg])


def _mlp_body(cnt_ref, cp_ref, g_ref, bias_ref, w1_ref, b1_ref, w2_ref,
              b2_ref, wout_ref, bout_ref, out_ref, *, T):
    s = pl.program_id(0)
    cnt = cnt_ref[s]
    nblk = (cnt + T - 1) // T
    g = g_ref[0]
    bias = bias_ref[0]
    w1 = w1_ref[...]
    b1 = b1_ref[...]
    w2 = w2_ref[...]
    b2 = b2_ref[...]
    wout = wout_ref[...]
    bout = bout_ref[...]

    def step(i, carry):
        for k in range(2):
            x = cp_ref[0, pl.ds((2 * i + k) * T, T), :]
            net = jnp.dot(x, g, preferred_element_type=_F32) + bias
            h = jnp.dot(jnp.maximum(net, 0.0), w1,
                        preferred_element_type=_F32) + b1
            h = jnp.dot(jnp.maximum(h, 0.0), w2,
                        preferred_element_type=_F32) + b2
            net = net + h
            occ = (jnp.sum(jnp.maximum(net, 0.0) * wout,
                           axis=1, keepdims=True) + bout)
            out_ref[0, :, pl.ds((2 * i + k) * T, T)] = (
                jax.nn.sigmoid(10.0 * occ).T)
        return carry

    lax.fori_loop(0, (nblk + 1) // 2, step, 0)


def _sc_scatter_body(sig_hbm, idx_hbm, imp_hbm, sig_v, idx_v, out_v, *, Q):
    c = lax.axis_index("c")
    s = lax.axis_index("s")
    seg = c * 8 + s

    @pl.when(s < 8)
    def _():
        pltpu.sync_copy(sig_hbm.at[seg], sig_v)
        pltpu.sync_copy(idx_hbm.at[seg], idx_v)
        zero = jnp.zeros((16,), _F32)

        def zer(j, carry):
            out_v[pl.ds(j * 16, 16)] = zero
            return carry

        lax.fori_loop(0, (Q + 16) // 16, zer, 0)

        def sca(j, carry):
            iv = idx_v[pl.ds(j * 16, 16)]
            sv = sig_v[pl.ds(j * 16, 16)]
            plsc.store_scatter(out_v, [iv], sv)
            return carry

        lax.fori_loop(0, Q // 16, sca, 0)
        pltpu.sync_copy(out_v.at[pl.ds(0, Q)], imp_hbm.at[seg])


def kernel(ray_points, translations, rotations, part_shape_features,
           points_mask, W_p, b_p, W_c, b_c, W1, b1, W2, b2, W_out, b_out):
    B, N, P, _ = ray_points.shape
    M = translations.shape[1]
    C = part_shape_features.shape[-1]
    H = W_p.shape[1]
    Q = N * P
    S = B * M

    a12 = pl.pallas_call(
        _prep1_body,
        out_shape=jax.ShapeDtypeStruct((S, 12), _F32),
    )(rotations.reshape(S, 4), translations.reshape(S, 3))

    abig = a12.reshape(S * 4, 3)
    a = a12.reshape(B, M, 4, 3).transpose(0, 2, 1, 3).reshape(B, 4, M * 3)

    g, bias = pl.pallas_call(
        _prep2_body,
        out_shape=(
            jax.ShapeDtypeStruct((S * 4, H), _F32),
            jax.ShapeDtypeStruct((S, H), _F32),
        ),
    )(abig, W_p, part_shape_features.reshape(S, C), W_c,
      b_c.reshape(1, H), b_p.reshape(1, H))

    pts = ray_points.reshape(B, Q, 3)
    paug = jnp.concatenate([pts, jnp.ones((B, Q, 1), _F32)], axis=-1)

    TP = 2048
    ptm = pl.pallas_call(
        _ptm_body,
        grid=(B, Q // TP),
        in_specs=[
            pl.BlockSpec((1, TP, 4), lambda b, i: (b, i, 0)),
            pl.BlockSpec((1, 4, M * 3), lambda b, i: (b, 0, 0)),
        ],
        out_specs=pl.BlockSpec((1, TP, M * 3), lambda b, i: (b, i, 0)),
        out_shape=jax.ShapeDtypeStruct((B, Q, M * 3), _F32),
        compiler_params=pltpu.CompilerParams(
            dimension_semantics=("parallel", "parallel")),
    )(paug, a)

    mask_t = (points_mask.reshape(B, Q, M).transpose(0, 2, 1)
              .reshape(S, Q).astype(_I32))
    paug_pad = jnp.concatenate(
        [paug, jnp.zeros((B, 16, 4), _F32)], axis=1).reshape(B, (Q + 16) * 4)

    mesh = plsc.VectorSubcoreMesh(core_axis_name="c", subcore_axis_name="s")
    idx, cp, cnt = pl.kernel(
        functools.partial(_sc_compact_body, Q=Q, M=M),
        out_type=(
            jax.ShapeDtypeStruct((S, Q), _I32),
            jax.ShapeDtypeStruct((S, Q * 4), _F32),
            jax.ShapeDtypeStruct((S, 16), _I32),
        ),
        mesh=mesh,
        scratch_types=[
            pltpu.VMEM((Q,), _I32),
            pltpu.VMEM(((Q + 16) * 4,), _F32),
            pltpu.VMEM((Q + 16,), _I32),
            pltpu.VMEM((Q * 4,), _F32),
            pltpu.VMEM((16,), _I32),
        ],
        compiler_params=pltpu.CompilerParams(needs_layout_passes=False),
    )(mask_t, paug_pad)

    counts = cnt[:, 0]

    T = 512
    sig = pl.pallas_call(
        functools.partial(_mlp_body, T=T),
        grid_spec=pltpu.PrefetchScalarGridSpec(
            num_scalar_prefetch=1,
            grid=(S,),
            in_specs=[
                pl.BlockSpec((1, Q, 4), lambda s, cnt: (s, 0, 0)),
                pl.BlockSpec((1, 4, H), lambda s, cnt: (s, 0, 0)),
                pl.BlockSpec((1, 1, H), lambda s, cnt: (s, 0, 0)),
                pl.BlockSpec((H, H), lambda s, cnt: (0, 0)),
                pl.BlockSpec((1, H), lambda s, cnt: (0, 0)),
                pl.BlockSpec((H, H), lambda s, cnt: (0, 0)),
                pl.BlockSpec((1, H), lambda s, cnt: (0, 0)),
                pl.BlockSpec((1, H), lambda s, cnt: (0, 0)),
                pl.BlockSpec((1, 1), lambda s, cnt: (0, 0)),
            ],
            out_specs=pl.BlockSpec((1, 1, Q), lambda s, cnt: (s, 0, 0)),
        ),
        out_shape=jax.ShapeDtypeStruct((S, 1, Q), _F32),
        compiler_params=pltpu.CompilerParams(
            dimension_semantics=("arbitrary",)),
    )(counts, cp.reshape(S, Q, 4), g.reshape(S, 4, H), bias.reshape(S, 1, H),
      W1, b1.reshape(1, H), W2, b2.reshape(1, H), W_out.reshape(1, H),
      b_out.reshape(1, 1))

    imp_t = pl.kernel(
        functools.partial(_sc_scatter_body, Q=Q),
        out_type=jax.ShapeDtypeStruct((S, Q), _F32),
        mesh=plsc.VectorSubcoreMesh(core_axis_name="c",
                                    subcore_axis_name="s"),
        scratch_types=[
            pltpu.VMEM((Q,), _F32),
            pltpu.VMEM((Q,), _I32),
            pltpu.VMEM((Q + 16,), _F32),
        ],
        compiler_params=pltpu.CompilerParams(needs_layout_passes=False),
    )(sig.reshape(S, Q), idx)

    implicit_field = (imp_t.reshape(B, M, Q).transpose(0, 2, 1)
                      .reshape(B, N, P, M))
    points_transformed = ptm.reshape(B, N, P, M * 3)
    return implicit_field, points_transformed


# speedup vs baseline: 1.9887x; 1.4490x over previous
import functools

import jax
import jax.numpy as jnp
from jax.experimental import pallas as pl
from jax.experimental.pallas import tpu as pltpu

_F32 = jnp.float32


def _body(pts_ref, rot_ref, tr_ref, feat_ref, mask_ref, wp_ref, wc_ref,
          bc_ref, bp_ref, w1_ref, b1_ref, w2_ref, b2_ref, wout_ref, bout_ref,
          ptm_ref, imp_ref, a_s, g_s, bias2_s, *, M, H):
    i = pl.program_id(1)

    @pl.when(i == 0)
    def _prep():
        q = rot_ref[0]
        norm = jnp.sqrt(jnp.sum(q * q, axis=1, keepdims=True))
        qn = q / jnp.maximum(norm, 1e-8)
        qw = qn[:, 0:1]
        qx = qn[:, 1:2]
        qy = qn[:, 2:3]
        qz = qn[:, 3:4]
        xx = qx * qx
        yy = qy * qy
        zz = qz * qz
        xy = qx * qy
        xz = qx * qz
        yz = qy * qz
        wx = qw * qx
        wy = qw * qy
        wz = qw * qz
        one = jnp.ones_like(qw)
        r00 = one - 2.0 * (yy + zz)
        r01 = 2.0 * (xy + wz)
        r02 = 2.0 * (xz - wy)
        r10 = 2.0 * (xy - wz)
        r11 = one - 2.0 * (xx + zz)
        r12 = 2.0 * (yz + wx)
        r20 = 2.0 * (xz + wy)
        r21 = 2.0 * (yz - wx)
        r22 = one - 2.0 * (xx + yy)
        t = tr_ref[0]
        tx = t[:, 0:1]
        ty = t[:, 1:2]
        tz = t[:, 2:3]
        c0 = -(r00 * tx + r01 * ty + r02 * tz)
        c1 = -(r10 * tx + r11 * ty + r12 * tz)
        c2 = -(r20 * tx + r21 * ty + r22 * tz)
        bias = (jnp.dot(feat_ref[0], wc_ref[...], preferred_element_type=_F32)
                + bc_ref[...] + bp_ref[...])
        wp = wp_ref[...]
        for m in range(M):
            s = slice(m, m + 1)
            row0 = jnp.concatenate([r00[s], r10[s], r20[s]], axis=1)
            row1 = jnp.concatenate([r01[s], r11[s], r21[s]], axis=1)
            row2 = jnp.concatenate([r02[s], r12[s], r22[s]], axis=1)
            crow = jnp.concatenate([c0[s], c1[s], c2[s]], axis=1)
            aaug = jnp.concatenate([row0, row1, row2, crow], axis=0)
            a_s[0:4, m * 3:(m + 1) * 3] = aaug
            gaug = jnp.dot(aaug, wp, preferred_element_type=_F32)
            g_s[0:3, m * H:(m + 1) * H] = gaug[0:3, :]
            bias2_s[s, :] = bias[s, :] + gaug[3:4, :]

    x = pts_ref[0]
    ptm_ref[0] = (jnp.dot(x, a_s[0:3, :], preferred_element_type=_F32)
                  + a_s[3:4, :])
    w1 = w1_ref[...]
    b1 = b1_ref[...]
    w2 = w2_ref[...]
    b2 = b2_ref[...]
    wout = wout_ref[...]
    bout = bout_ref[...]
    occ_cols = []
    for m in range(M):
        net = (jnp.dot(x, g_s[0:3, m * H:(m + 1) * H],
                       preferred_element_type=_F32) + bias2_s[m:m + 1, :])
        h = jnp.dot(jnp.maximum(net, 0.0), w1,
                    preferred_element_type=_F32) + b1
        h = jnp.dot(jnp.maximum(h, 0.0), w2,
                    preferred_element_type=_F32) + b2
        net = net + h
        occ_cols.append(
            jnp.sum(jnp.maximum(net, 0.0) * wout, axis=1, keepdims=True)
            + bout)
    occ = jnp.concatenate(occ_cols, axis=1)
    imp_ref[0] = jnp.where(mask_ref[0], jax.nn.sigmoid(10.0 * occ), 0.0)


def kernel(ray_points, translations, rotations, part_shape_features,
           points_mask, W_p, b_p, W_c, b_c, W1, b1, W2, b2, W_out, b_out):
    B, N, P, _ = ray_points.shape
    M = translations.shape[1]
    C = part_shape_features.shape[-1]
    H = W_p.shape[1]
    Q = N * P

    T = 512
    grid = (B, Q // T)
    ptm, imp = pl.pallas_call(
        functools.partial(_body, M=M, H=H),
        grid=grid,
        in_specs=[
            pl.BlockSpec((1, T, 3), lambda b, i: (b, i, 0)),
            pl.BlockSpec((1, M, 4), lambda b, i: (b, 0, 0)),
            pl.BlockSpec((1, M, 3), lambda b, i: (b, 0, 0)),
            pl.BlockSpec((1, M, C), lambda b, i: (b, 0, 0)),
            pl.BlockSpec((1, T, M), lambda b, i: (b, i, 0)),
            pl.BlockSpec((3, H), lambda b, i: (0, 0)),
            pl.BlockSpec((C, H), lambda b, i: (0, 0)),
            pl.BlockSpec((1, H), lambda b, i: (0, 0)),
            pl.BlockSpec((1, H), lambda b, i: (0, 0)),
            pl.BlockSpec((H, H), lambda b, i: (0, 0)),
            pl.BlockSpec((1, H), lambda b, i: (0, 0)),
            pl.BlockSpec((H, H), lambda b, i: (0, 0)),
            pl.BlockSpec((1, H), lambda b, i: (0, 0)),
            pl.BlockSpec((1, H), lambda b, i: (0, 0)),
            pl.BlockSpec((1, 1), lambda b, i: (0, 0)),
        ],
        out_specs=[
            pl.BlockSpec((1, T, M * 3), lambda b, i: (b, i, 0)),
            pl.BlockSpec((1, T, M), lambda b, i: (b, i, 0)),
        ],
        out_shape=(
            jax.ShapeDtypeStruct((B, Q, M * 3), _F32),
            jax.ShapeDtypeStruct((B, Q, M), _F32),
        ),
        scratch_shapes=[
            pltpu.VMEM((8, M * 3), _F32),
            pltpu.VMEM((8, M * H), _F32),
            pltpu.VMEM((M, H), _F32),
        ],
        compiler_params=pltpu.CompilerParams(
            dimension_semantics=("arbitrary", "arbitrary")),
    )(
        ray_points.reshape(B, Q, 3), rotations, translations,
        part_shape_features, points_mask.reshape(B, Q, M), W_p, W_c,
        b_c.reshape(1, H), b_p.reshape(1, H), W1, b1.reshape(1, H),
        W2, b2.reshape(1, H), W_out.reshape(1, H), b_out.reshape(1, 1),
    )

    implicit_field = imp.reshape(B, N, P, M)
    points_transformed = ptm.reshape(B, N, P, M * 3)
    return implicit_field, points_transformed
